# baseline (speedup 1.0000x reference)
import jax
import jax.numpy as jnp
from jax import lax
from jax.experimental import pallas as pl
from jax.experimental.pallas import tpu as pltpu
from jax.experimental.pallas import tpu_sc as plsc

_B, _N = 1024, 65536
_K = 32
_STRIDE = _N // _K
_LANES = 16
_K_SC = 16


def _sc_body(x_hbm, out_hbm, buf, out_v, sem):
    nc = plsc.get_sparse_core_info().num_cores
    wid = lax.axis_index("s") * nc + lax.axis_index("c")
    rows = _B // (nc * 16)
    r0 = wid * rows

    copies = [
        pltpu.make_async_copy(
            x_hbm.at[pl.ds(r0, rows), pl.ds(i * _STRIDE, 128)],
            buf.at[i],
            sem,
        )
        for i in range(_K_SC)
    ]
    for cp in copies:
        cp.start()
    for cp in copies:
        cp.wait()

    lane = lax.iota(jnp.int32, _LANES)
    zeros = jnp.zeros((_LANES,), jnp.int32)

    def extract(b, carry):
        out_v[b, :] = plsc.load_gather(
            buf, [lane, jnp.full((_LANES,), b, jnp.int32), zeros]
        )
        return carry

    lax.fori_loop(0, rows, extract, 0)

    pltpu.sync_copy(out_v, out_hbm.at[pl.ds(r0, rows), :])


def _sc_call(x):
    mesh = plsc.VectorSubcoreMesh(core_axis_name="c", subcore_axis_name="s")
    return pl.kernel(
        _sc_body,
        mesh=mesh,
        out_type=jax.ShapeDtypeStruct((_B, _K_SC), jnp.float32),
        scratch_types=[
            pltpu.VMEM((_K_SC, _B // 32, 128), jnp.float32),
            pltpu.VMEM((_B // 32, _K_SC), jnp.float32),
            pltpu.SemaphoreType.DMA,
        ],
        compiler_params=pltpu.CompilerParams(needs_layout_passes=False),
    )(x)


def _tc_body(*refs):
    o_ref = refs[-1]
    o_ref[...] = jnp.concatenate([r[:, 0:1] for r in refs[:-1]], axis=1)


def _tc_spec(i):
    return pl.BlockSpec((1024, 128), lambda _, i=i: (0, 16 * i))


def _tc_call(x):
    n = _K - _K_SC
    return pl.pallas_call(
        _tc_body,
        grid=(1,),
        in_specs=[_tc_spec(i) for i in range(_K_SC, _K)],
        out_specs=pl.BlockSpec((1024, n), lambda _: (0, 0)),
        out_shape=jax.ShapeDtypeStruct((_B, n), jnp.float32),
    )(*([x] * n))


def kernel(x):
    out_sc = _sc_call(x)
    out_tc = _tc_call(x)
    return jnp.concatenate([out_sc, out_tc], axis=1)


# speedup vs baseline: 1.9299x; 1.9299x over previous
import jax
import jax.numpy as jnp
from jax.experimental import pallas as pl


def _body(*refs):
    o_ref = refs[-1]
    o_ref[...] = jnp.concatenate([r[:, 0:1] for r in refs[:-1]], axis=1)


def _spec(i):
    return pl.BlockSpec((1024, 128), lambda _, i=i: (0, 16 * i))


def kernel(x):
    return pl.pallas_call(
        _body,
        grid=(1,),
        in_specs=[_spec(i) for i in range(32)],
        out_specs=pl.BlockSpec((1024, 32), lambda _: (0, 0)),
        out_shape=jax.ShapeDtypeStruct((1024, 32), jnp.float32),
    )(*([x] * 32))


# speedup vs baseline: 1.9334x; 1.0018x over previous
import jax
import jax.numpy as jnp
from jax.experimental import pallas as pl


def _body(*refs):
    o_ref = refs[-1]
    ins = refs[:-1]
    halves = []
    for h in range(2):
        halves.append(
            jnp.concatenate([r[:, 0:1] for r in ins[32 * h : 32 * h + 32]], axis=1)
        )
    o_ref[...] = jnp.concatenate(halves, axis=0)


def _spec(j):
    h, i = divmod(j, 32)
    return pl.BlockSpec((512, 128), lambda _, h=h, i=i: (h, 16 * i))


def kernel(x):
    return pl.pallas_call(
        _body,
        grid=(1,),
        in_specs=[_spec(j) for j in range(64)],
        out_specs=pl.BlockSpec((1024, 32), lambda _: (0, 0)),
        out_shape=jax.ShapeDtypeStruct((1024, 32), jnp.float32),
    )(*([x] * 64))


# speedup vs baseline: 1.9352x; 1.0009x over previous
import jax
import jax.numpy as jnp
from jax.experimental import pallas as pl


def _body(*refs):
    o_ref = refs[-1]
    o_ref[:, 0:32] = jnp.concatenate([r[:, 0:1] for r in refs[:-1]], axis=1)


def _spec(i):
    return pl.BlockSpec((1024, 128), lambda _, i=i: (0, 16 * i))


def kernel(x):
    out = pl.pallas_call(
        _body,
        grid=(1,),
        in_specs=[_spec(i) for i in range(32)],
        out_specs=pl.BlockSpec((1024, 128), lambda _: (0, 0)),
        out_shape=jax.ShapeDtypeStruct((1024, 128), jnp.float32),
    )(*([x] * 32))
    return out[:, :32]
